# baseline (speedup 1.0000x reference)
import functools
import math

import jax
import jax.numpy as jnp
from jax import lax
from jax.experimental import pallas as pl
from jax.experimental.pallas import tpu as pltpu
from jax.experimental.pallas import tpu_sc as plsc

NE = 8
TOPK = 2
CAP_F = 1.25
LANES = 128
NC = 2
NS = 16
NW = NC * NS
SCL = 16


def _pack_bf16(a):
    h2 = a.shape[1] // 2
    ab = a.astype(jnp.bfloat16)
    lo = lax.bitcast_convert_type(ab[:, :h2], jnp.uint16).astype(jnp.uint32)
    hi = lax.bitcast_convert_type(ab[:, h2:], jnp.uint16).astype(jnp.uint32)
    return lax.bitcast_convert_type(lo | (hi << 16), jnp.int32)


def _unpack_bf16(pk):
    u = lax.bitcast_convert_type(pk, jnp.uint32)
    lo = lax.bitcast_convert_type((u & 0xFFFF).astype(jnp.uint16),
                                  jnp.bfloat16)
    hi = lax.bitcast_convert_type((u >> 16).astype(jnp.uint16), jnp.bfloat16)
    f32 = jnp.float32
    return jnp.concatenate([lo.astype(f32), hi.astype(f32)], axis=1)


def _router_body(cap, T, x_ref, rw_ref, ltri_ref, dsts_ref, dstr_ref,
                 w0_ref, w1_ref, aux_ref, z_ref, xpk_ref):
    f32 = jnp.float32
    x = x_ref[...]
    xpk_ref[...] = _pack_bf16(x)
    logits = jnp.dot(x, rw_ref[...], preferred_element_type=f32)
    lane = lax.broadcasted_iota(jnp.int32, (T, LANES), 1)
    valid_lane = lane < NE
    lm = jnp.where(valid_lane, logits, -1e30)
    mx = jnp.max(lm, axis=1, keepdims=True)
    ex = jnp.where(valid_lane, jnp.exp(lm - mx), 0.0)
    se = jnp.sum(ex, axis=1, keepdims=True)
    probs = ex / se

    m0 = jnp.max(probs, axis=1, keepdims=True)
    i0 = jnp.min(jnp.where((probs == m0) & valid_lane, lane, NE),
                 axis=1, keepdims=True)
    sel0 = lane == i0
    p2 = jnp.where(sel0 | ~valid_lane, -1.0, probs)
    m1 = jnp.max(p2, axis=1, keepdims=True)
    i1 = jnp.min(jnp.where((p2 == m1) & valid_lane, lane, NE),
                 axis=1, keepdims=True)
    sel1 = lane == i1

    B = 128
    L = ltri_ref[...]
    mask0 = sel0.astype(f32)
    mask1 = sel1.astype(f32)
    carry0 = jnp.zeros((1, LANES), f32)
    carry1 = jnp.zeros((1, LANES), f32)
    blocks0 = []
    blocks1 = []
    for b in range(T // B):
        mb0 = mask0[b * B:(b + 1) * B, :]
        mb1 = mask1[b * B:(b + 1) * B, :]
        cb0 = jnp.dot(L, mb0, preferred_element_type=f32) + carry0
        cb1 = jnp.dot(L, mb1, preferred_element_type=f32) + carry1
        carry0 = cb0[B - 1:B, :]
        carry1 = cb1[B - 1:B, :]
        blocks0.append(cb0)
        blocks1.append(cb1)
    c0 = jnp.concatenate(blocks0, axis=0)
    c1 = jnp.concatenate(blocks1, axis=0)
    tot0 = carry0
    tot1 = carry1

    s0 = jnp.sum(jnp.where(sel0, c0 - 1.0, 0.0), axis=1, keepdims=True)
    s1 = jnp.sum(jnp.where(sel1, c1 - 1.0 + tot0, 0.0), axis=1, keepdims=True)
    s0i = s0.astype(jnp.int32)
    s1i = s1.astype(jnp.int32)
    v0 = s0i < cap
    v1 = s1i < cap
    tv0 = jnp.sum(jnp.where(sel0, probs, 0.0), axis=1, keepdims=True)
    tv1 = jnp.sum(jnp.where(sel1, probs, 0.0), axis=1, keepdims=True)

    trash = NE * cap
    d0 = jnp.where(v0, i0 * cap + s0i, trash)
    d1 = jnp.where(v1, i1 * cap + s1i, trash)
    dsts_ref[:, 0:1] = d0
    dsts_ref[:, 1:2] = d1
    dstr_ref[:, 0:1] = jnp.where(v0, d0, 0)
    dstr_ref[:, 1:2] = jnp.where(v1, d1, 0)
    ones16 = jnp.ones((1, SCL), f32)
    w0_ref[...] = jnp.where(v0, tv0, 0.0) * ones16
    w1_ref[...] = jnp.where(v1, tv1, 0.0) * ones16

    count = jnp.minimum(jnp.float32(cap), tot0 + tot1)
    mean_prob = jnp.sum(probs, axis=0, keepdims=True) / T
    aux_ref[...] = NE * jnp.sum(mean_prob * count, keepdims=True) / T
    lse = mx + jnp.log(se)
    z_ref[...] = jnp.sum(lse * lse, keepdims=True).reshape(1, 1) / T


def _router_call(x_flat, rw_pad, ltri, cap):
    T = x_flat.shape[0]
    f32 = jnp.float32
    return pl.pallas_call(
        functools.partial(_router_body, cap, T),
        out_shape=[
            jax.ShapeDtypeStruct((T, 2), jnp.int32),
            jax.ShapeDtypeStruct((T, 2), jnp.int32),
            jax.ShapeDtypeStruct((T, SCL), f32),
            jax.ShapeDtypeStruct((T, SCL), f32),
            jax.ShapeDtypeStruct((1, 1), f32),
            jax.ShapeDtypeStruct((1, 1), f32),
            jax.ShapeDtypeStruct((T, x_flat.shape[1] // 2), jnp.int32),
        ],
    )(x_flat, rw_pad, ltri)


SROW = 32


def _build_slot_map(T, dstbuf, mapb, base_slot, spt, init_val, token_only):
    i32 = jnp.int32
    nr = spt // SROW
    for r in range(nr):
        for h in range(SROW // SCL):
            mapb[r, pl.ds(h * SCL, SCL)] = jnp.full((SCL,), init_val, i32)
    pos16 = lax.iota(i32, SCL)

    def step(i, c):
        v = dstbuf[pl.ds(i * SCL, SCL)]
        pos = pos16 + i * SCL
        m = (v >= base_slot) & (v < base_slot + spt)
        s = jnp.where(m, v - base_slot, 0)
        row = lax.shift_right_logical(s, 5)
        col = s & (SROW - 1)
        tok = lax.shift_right_logical(pos, 1)
        val = tok if token_only else (pos & 1) * T + tok
        plsc.store_scatter(mapb, [row, col], val, mask=m)
        return c

    lax.fori_loop(0, 2 * T // SCL, step, 0)


def _dispatch_body(T, HP, cap, x_hbm, dst_hbm, disp_hbm,
                   dstbuf, mapb, b0, b1, b2, b3, semg, semo):
    spt = NE * cap // NW
    nr = spt // SROW
    wid = lax.axis_index("s") * NC + lax.axis_index("c")
    base_slot = wid * spt
    pltpu.sync_copy(dst_hbm, dstbuf)
    _build_slot_map(T, dstbuf, mapb, base_slot, spt, 0, token_only=True)
    bufs = [b0, b1, b2, b3]

    def gather(c):
        return pltpu.async_copy(x_hbm.at[mapb.at[c]], bufs[c % 4], semg)

    gat = [None] * nr
    sto = [None] * nr
    drained = set()
    for c in range(min(2, nr)):
        gat[c] = gather(c)
    for c in range(nr):
        k = c + 2
        if k < nr:
            if k >= 4:
                sto[k - 4].wait()
                drained.add(k - 4)
            gat[k] = gather(k)
        gat[c].wait()
        sto[c] = pltpu.async_copy(
            bufs[c % 4], disp_hbm.at[pl.ds(base_slot + c * SROW, SROW)], semo)
    for c in range(nr):
        if c not in drained:
            sto[c].wait()


def _dispatch_call(xpk, dsts_flat, cap):
    T, HP = xpk.shape
    spt = NE * cap // NW
    mesh = plsc.VectorSubcoreMesh(core_axis_name="c", subcore_axis_name="s",
                                  num_cores=NC, num_subcores=NS)
    return pl.kernel(
        functools.partial(_dispatch_body, T, HP, cap),
        out_type=jax.ShapeDtypeStruct((NE * cap, HP), jnp.int32),
        mesh=mesh,
        compiler_params=pltpu.CompilerParams(needs_layout_passes=False),
        scratch_types=[
            pltpu.VMEM((2 * T,), jnp.int32),
            pltpu.VMEM((spt // SROW, SROW), jnp.int32),
            pltpu.VMEM((SROW, HP), jnp.int32),
            pltpu.VMEM((SROW, HP), jnp.int32),
            pltpu.VMEM((SROW, HP), jnp.int32),
            pltpu.VMEM((SROW, HP), jnp.int32),
            pltpu.SemaphoreType.DMA,
            pltpu.SemaphoreType.DMA,
        ],
    )(xpk, dsts_flat)


def _ffn_body(ni, xpk_ref, gw_ref, gb_ref, uw_ref, ub_ref, dw_ref, db_ref,
              out_ref, acc_ref, xs_ref):
    i = pl.program_id(1)
    f32 = jnp.float32

    @pl.when(i == 0)
    def _():
        xs_ref[...] = _unpack_bf16(xpk_ref[...])

    x = xs_ref[...]
    g = jnp.dot(x, gw_ref[0], preferred_element_type=f32) + gb_ref[0]
    u = jnp.dot(x, uw_ref[0], preferred_element_type=f32) + ub_ref[0]
    h = g * (1.0 / (1.0 + jnp.exp(-g))) * u
    part = jnp.dot(h, dw_ref[0], preferred_element_type=f32)

    @pl.when(i == 0)
    def _():
        acc_ref[...] = part + db_ref[0]

    @pl.when(i > 0)
    def _():
        acc_ref[...] = acc_ref[...] + part

    @pl.when(i == ni - 1)
    def _():
        out_ref[...] = _pack_bf16(acc_ref[...])


def _ffn_call(disp_pk, gate_w, gate_b, up_w, up_b, down_w, down_b, cap):
    HP = disp_pk.shape[1]
    H = 2 * HP
    I = gate_w.shape[2]
    NI = 4
    TI = I // NI
    f32 = jnp.float32
    return pl.pallas_call(
        functools.partial(_ffn_body, NI),
        grid=(NE, NI),
        scratch_shapes=[pltpu.VMEM((cap, H), f32),
                        pltpu.VMEM((cap, H), f32)],
        in_specs=[
            pl.BlockSpec((cap, HP), lambda e, i: (e, 0)),
            pl.BlockSpec((1, H, TI), lambda e, i: (e, 0, i)),
            pl.BlockSpec((1, 1, TI), lambda e, i: (e, 0, i)),
            pl.BlockSpec((1, H, TI), lambda e, i: (e, 0, i)),
            pl.BlockSpec((1, 1, TI), lambda e, i: (e, 0, i)),
            pl.BlockSpec((1, TI, H), lambda e, i: (e, i, 0)),
            pl.BlockSpec((1, 1, H), lambda e, i: (e, 0, 0)),
        ],
        out_specs=pl.BlockSpec((cap, HP), lambda e, i: (e, 0)),
        out_shape=jax.ShapeDtypeStruct((NE * cap, HP), jnp.int32),
    )(disp_pk, gate_w, gate_b[:, None, :], up_w, up_b[:, None, :],
      down_w, down_b[:, None, :])


def _combine_body(T, HP, cap, ffn_hbm, dst_hbm, g_hbm,
                  dstbuf, mapb, b0, b1, b2, b3, semg, semo):
    spt = NE * cap // NW
    nr = spt // SROW
    wid = lax.axis_index("s") * NC + lax.axis_index("c")
    base_slot = wid * spt
    pltpu.sync_copy(dst_hbm, dstbuf)
    _build_slot_map(T, dstbuf, mapb, base_slot, spt, 2 * T, token_only=False)
    bufs = [b0, b1, b2, b3]

    def load(c):
        return pltpu.async_copy(
            ffn_hbm.at[pl.ds(base_slot + c * SROW, SROW)], bufs[c % 4], semg)

    lod = [None] * nr
    sto = [None] * nr
    drained = set()
    for c in range(min(2, nr)):
        lod[c] = load(c)
    for c in range(nr):
        k = c + 2
        if k < nr:
            if k >= 4:
                sto[k - 4].wait()
                drained.add(k - 4)
            lod[k] = load(k)
        lod[c].wait()
        sto[c] = pltpu.async_copy(bufs[c % 4], g_hbm.at[mapb.at[c]], semo)
    for c in range(nr):
        if c not in drained:
            sto[c].wait()


def _combine_call(ffn_pk, dsts_flat, cap, T):
    HP = ffn_pk.shape[1]
    spt = NE * cap // NW
    mesh = plsc.VectorSubcoreMesh(core_axis_name="c", subcore_axis_name="s",
                                  num_cores=NC, num_subcores=NS)
    return pl.kernel(
        functools.partial(_combine_body, T, HP, cap),
        out_type=jax.ShapeDtypeStruct((2 * T + 8, HP), jnp.int32),
        mesh=mesh,
        compiler_params=pltpu.CompilerParams(needs_layout_passes=False),
        scratch_types=[
            pltpu.VMEM((2 * T,), jnp.int32),
            pltpu.VMEM((spt // SROW, SROW), jnp.int32),
            pltpu.VMEM((SROW, HP), jnp.int32),
            pltpu.VMEM((SROW, HP), jnp.int32),
            pltpu.VMEM((SROW, HP), jnp.int32),
            pltpu.VMEM((SROW, HP), jnp.int32),
            pltpu.SemaphoreType.DMA,
            pltpu.SemaphoreType.DMA,
        ],
    )(ffn_pk, dsts_flat)


def _wsum_body(g0_ref, g1_ref, w0_ref, w1_ref, out_ref):
    w0 = w0_ref[:, 0:1]
    w1 = w1_ref[:, 0:1]
    a0 = jnp.where(w0 > 0, _unpack_bf16(g0_ref[...]) * w0, 0.0)
    a1 = jnp.where(w1 > 0, _unpack_bf16(g1_ref[...]) * w1, 0.0)
    out_ref[...] = a0 + a1


def _wsum_call(gpk, w0x, w1x, T):
    HP = gpk.shape[1]
    H = 2 * HP
    TB = 512
    nb = T // TB
    return pl.pallas_call(
        _wsum_body,
        grid=(nb,),
        in_specs=[
            pl.BlockSpec((TB, HP), lambda t: (t, 0)),
            pl.BlockSpec((TB, HP), lambda t, n=nb: (t + n, 0)),
            pl.BlockSpec((TB, SCL), lambda t: (t, 0)),
            pl.BlockSpec((TB, SCL), lambda t: (t, 0)),
        ],
        out_specs=pl.BlockSpec((TB, H), lambda t: (t, 0)),
        out_shape=jax.ShapeDtypeStruct((T, H), jnp.float32),
    )(gpk, gpk, w0x, w1x)


def kernel(x, router_w, gate_w, gate_b, up_w, up_b, down_w, down_b):
    bsz, seq, H = x.shape
    T = bsz * seq
    cap = max(1, math.ceil(CAP_F * T / NE))
    x_flat = x.reshape(T, H)

    rw_pad = jnp.pad(router_w, ((0, 0), (0, LANES - NE)))
    ltri = jnp.tril(jnp.ones((128, 128), jnp.float32))

    dsts, dstr, w0x, w1x, aux, z, xpk = _router_call(x_flat, rw_pad, ltri, cap)
    dsts_flat = dsts.reshape(-1)

    disp_pk = _dispatch_call(xpk, dsts_flat, cap)
    ffn_pk = _ffn_call(disp_pk, gate_w, gate_b, up_w, up_b, down_w, down_b,
                       cap)
    gpk = _combine_call(ffn_pk, dsts_flat, cap, T)
    out = _wsum_call(gpk, w0x, w1x, T)

    return (out.reshape(bsz, seq, H), aux.reshape(()), z.reshape(()))


# speedup vs baseline: 1.0089x; 1.0089x over previous
import functools
import math

import numpy as np

import jax
import jax.numpy as jnp
from jax import lax
from jax.experimental import pallas as pl
from jax.experimental.pallas import tpu as pltpu
from jax.experimental.pallas import tpu_sc as plsc

NE = 8
TOPK = 2
CAP_F = 1.25
LANES = 128
NC = 2
NS = 16
NW = NC * NS
SCL = 16


def _pack_bf16(a):
    h2 = a.shape[1] // 2
    ab = a.astype(jnp.bfloat16)
    lo = lax.bitcast_convert_type(ab[:, :h2], jnp.uint16).astype(jnp.uint32)
    hi = lax.bitcast_convert_type(ab[:, h2:], jnp.uint16).astype(jnp.uint32)
    return lax.bitcast_convert_type(lo | (hi << 16), jnp.int32)


def _unpack_bf16(pk):
    u = lax.bitcast_convert_type(pk, jnp.uint32)
    lo = lax.bitcast_convert_type((u & 0xFFFF).astype(jnp.uint16),
                                  jnp.bfloat16)
    hi = lax.bitcast_convert_type((u >> 16).astype(jnp.uint16), jnp.bfloat16)
    f32 = jnp.float32
    return jnp.concatenate([lo.astype(f32), hi.astype(f32)], axis=1)


def _router_body(cap, T, x_ref, rw_ref, ltri_ref, dsts_ref,
                 w0_ref, w1_ref, aux_ref, z_ref, xpk_ref):
    f32 = jnp.float32
    x = x_ref[...]
    xpk_ref[...] = _pack_bf16(x)
    logits = jnp.dot(x, rw_ref[...], preferred_element_type=f32)
    lane = lax.broadcasted_iota(jnp.int32, (T, LANES), 1)
    valid_lane = lane < NE
    lm = jnp.where(valid_lane, logits, -1e30)
    mx = jnp.max(lm, axis=1, keepdims=True)
    ex = jnp.where(valid_lane, jnp.exp(lm - mx), 0.0)
    se = jnp.sum(ex, axis=1, keepdims=True)
    probs = ex / se

    m0 = jnp.max(probs, axis=1, keepdims=True)
    i0 = jnp.min(jnp.where((probs == m0) & valid_lane, lane, NE),
                 axis=1, keepdims=True)
    sel0 = lane == i0
    p2 = jnp.where(sel0 | ~valid_lane, -1.0, probs)
    m1 = jnp.max(p2, axis=1, keepdims=True)
    i1 = jnp.min(jnp.where((p2 == m1) & valid_lane, lane, NE),
                 axis=1, keepdims=True)
    sel1 = lane == i1

    B = 128
    L = ltri_ref[...]
    mask0 = sel0.astype(f32)
    mask1 = sel1.astype(f32)
    carry0 = jnp.zeros((1, LANES), f32)
    carry1 = jnp.zeros((1, LANES), f32)
    blocks0 = []
    blocks1 = []
    for b in range(T // B):
        mb0 = mask0[b * B:(b + 1) * B, :]
        mb1 = mask1[b * B:(b + 1) * B, :]
        cb0 = jnp.dot(L, mb0, preferred_element_type=f32) + carry0
        cb1 = jnp.dot(L, mb1, preferred_element_type=f32) + carry1
        carry0 = cb0[B - 1:B, :]
        carry1 = cb1[B - 1:B, :]
        blocks0.append(cb0)
        blocks1.append(cb1)
    c0 = jnp.concatenate(blocks0, axis=0)
    c1 = jnp.concatenate(blocks1, axis=0)
    tot0 = carry0
    tot1 = carry1

    s0 = jnp.sum(jnp.where(sel0, c0 - 1.0, 0.0), axis=1, keepdims=True)
    s1 = jnp.sum(jnp.where(sel1, c1 - 1.0 + tot0, 0.0), axis=1, keepdims=True)
    s0i = s0.astype(jnp.int32)
    s1i = s1.astype(jnp.int32)
    v0 = s0i < cap
    v1 = s1i < cap
    tv0 = jnp.sum(jnp.where(sel0, probs, 0.0), axis=1, keepdims=True)
    tv1 = jnp.sum(jnp.where(sel1, probs, 0.0), axis=1, keepdims=True)

    trash = NE * cap
    d0 = jnp.where(v0, i0 * cap + s0i, trash)
    d1 = jnp.where(v1, i1 * cap + s1i, trash)
    dsts_ref[:, 0:1] = d0
    dsts_ref[:, 1:2] = d1
    ones16 = jnp.ones((1, SCL), f32)
    w0_ref[...] = jnp.where(v0, tv0, 0.0) * ones16
    w1_ref[...] = jnp.where(v1, tv1, 0.0) * ones16

    count = jnp.minimum(jnp.float32(cap), tot0 + tot1)
    mean_prob = jnp.sum(probs, axis=0, keepdims=True) / T
    aux_ref[...] = NE * jnp.sum(mean_prob * count, keepdims=True) / T
    lse = mx + jnp.log(se)
    z_ref[...] = jnp.sum(lse * lse, keepdims=True).reshape(1, 1) / T


def _router_call(x_flat, rw_pad, ltri, cap):
    T = x_flat.shape[0]
    f32 = jnp.float32
    return pl.pallas_call(
        functools.partial(_router_body, cap, T),
        out_shape=[
            jax.ShapeDtypeStruct((T, 2), jnp.int32),
            jax.ShapeDtypeStruct((T, SCL), f32),
            jax.ShapeDtypeStruct((T, SCL), f32),
            jax.ShapeDtypeStruct((1, 1), f32),
            jax.ShapeDtypeStruct((1, 1), f32),
            jax.ShapeDtypeStruct((T, x_flat.shape[1] // 2), jnp.int32),
        ],
    )(x_flat, rw_pad, ltri)


SROW = 32


def _build_slot_map(T, dstbuf, mapb, base_slot, spt, init_val, token_only):
    i32 = jnp.int32
    nr = spt // SROW
    for r in range(nr):
        for h in range(SROW // SCL):
            mapb[r, pl.ds(h * SCL, SCL)] = jnp.full((SCL,), init_val, i32)
    pos16 = lax.iota(i32, SCL)

    def step(i, c):
        v = dstbuf[pl.ds(i * SCL, SCL)]
        pos = pos16 + i * SCL
        m = (v >= base_slot) & (v < base_slot + spt)
        s = jnp.where(m, v - base_slot, 0)
        row = lax.shift_right_logical(s, 5)
        col = s & (SROW - 1)
        tok = lax.shift_right_logical(pos, 1)
        val = tok if token_only else (pos & 1) * T + tok
        plsc.store_scatter(mapb, [row, col], val, mask=m)
        return c

    lax.fori_loop(0, 2 * T // SCL, step, 0)


def _dispatch_body(T, HP, cap, x_hbm, dst_hbm, disp_hbm,
                   dstbuf, mapb, b0, b1, b2, b3, semg, semo):
    spt = NE * cap // NW
    nr = spt // SROW
    wid = lax.axis_index("s") * NC + lax.axis_index("c")
    base_slot = wid * spt
    pltpu.sync_copy(dst_hbm, dstbuf)
    _build_slot_map(T, dstbuf, mapb, base_slot, spt, 0, token_only=True)
    bufs = [b0, b1, b2, b3]

    def gather(c):
        return pltpu.async_copy(x_hbm.at[mapb.at[c]], bufs[c % 4], semg)

    gat = [None] * nr
    sto = [None] * nr
    drained = set()
    for c in range(min(2, nr)):
        gat[c] = gather(c)
    for c in range(nr):
        k = c + 2
        if k < nr:
            if k >= 4:
                sto[k - 4].wait()
                drained.add(k - 4)
            gat[k] = gather(k)
        gat[c].wait()
        sto[c] = pltpu.async_copy(
            bufs[c % 4], disp_hbm.at[pl.ds(base_slot + c * SROW, SROW)], semo)
    for c in range(nr):
        if c not in drained:
            sto[c].wait()


def _dispatch_call(xpk, dsts_flat, cap):
    T, HP = xpk.shape
    spt = NE * cap // NW
    mesh = plsc.VectorSubcoreMesh(core_axis_name="c", subcore_axis_name="s",
                                  num_cores=NC, num_subcores=NS)
    return pl.kernel(
        functools.partial(_dispatch_body, T, HP, cap),
        out_type=jax.ShapeDtypeStruct((NE * cap, HP), jnp.int32),
        mesh=mesh,
        compiler_params=pltpu.CompilerParams(needs_layout_passes=False),
        scratch_types=[
            pltpu.VMEM((2 * T,), jnp.int32),
            pltpu.VMEM((spt // SROW, SROW), jnp.int32),
            pltpu.VMEM((SROW, HP), jnp.int32),
            pltpu.VMEM((SROW, HP), jnp.int32),
            pltpu.VMEM((SROW, HP), jnp.int32),
            pltpu.VMEM((SROW, HP), jnp.int32),
            pltpu.SemaphoreType.DMA,
            pltpu.SemaphoreType.DMA,
        ],
    )(xpk, dsts_flat)


def _ffn_body(ni, xpk_ref, gw_ref, gb_ref, uw_ref, ub_ref, dw_ref, db_ref,
              out_ref, acc_ref, xs_ref):
    i = pl.program_id(1)
    f32 = jnp.float32

    @pl.when(i == 0)
    def _():
        xs_ref[...] = _unpack_bf16(xpk_ref[...])

    x = xs_ref[...]
    g = jnp.dot(x, gw_ref[0], preferred_element_type=f32) + gb_ref[0]
    u = jnp.dot(x, uw_ref[0], preferred_element_type=f32) + ub_ref[0]
    h = g * (1.0 / (1.0 + jnp.exp(-g))) * u
    part = jnp.dot(h, dw_ref[0], preferred_element_type=f32)

    @pl.when(i == 0)
    def _():
        acc_ref[...] = part + db_ref[0]

    @pl.when(i > 0)
    def _():
        acc_ref[...] = acc_ref[...] + part

    @pl.when(i == ni - 1)
    def _():
        out_ref[...] = _pack_bf16(acc_ref[...])


def _ffn_call(disp_pk, gate_w, gate_b, up_w, up_b, down_w, down_b, cap):
    HP = disp_pk.shape[1]
    H = 2 * HP
    I = gate_w.shape[2]
    NI = 4
    TI = I // NI
    f32 = jnp.float32
    return pl.pallas_call(
        functools.partial(_ffn_body, NI),
        grid=(NE, NI),
        scratch_shapes=[pltpu.VMEM((cap, H), f32),
                        pltpu.VMEM((cap, H), f32)],
        in_specs=[
            pl.BlockSpec((cap, HP), lambda e, i: (e, 0)),
            pl.BlockSpec((1, H, TI), lambda e, i: (e, 0, i)),
            pl.BlockSpec((1, 1, TI), lambda e, i: (e, 0, i)),
            pl.BlockSpec((1, H, TI), lambda e, i: (e, 0, i)),
            pl.BlockSpec((1, 1, TI), lambda e, i: (e, 0, i)),
            pl.BlockSpec((1, TI, H), lambda e, i: (e, i, 0)),
            pl.BlockSpec((1, 1, H), lambda e, i: (e, 0, 0)),
        ],
        out_specs=pl.BlockSpec((cap, HP), lambda e, i: (e, 0)),
        out_shape=jax.ShapeDtypeStruct((NE * cap, HP), jnp.int32),
    )(disp_pk, gate_w, gate_b[:, None, :], up_w, up_b[:, None, :],
      down_w, down_b[:, None, :])


def _combine_body(T, HP, cap, ffn_hbm, dst_hbm, g_hbm,
                  dstbuf, mapb, b0, b1, b2, b3, semg, semo):
    spt = NE * cap // NW
    nr = spt // SROW
    wid = lax.axis_index("s") * NC + lax.axis_index("c")
    base_slot = wid * spt
    pltpu.sync_copy(dst_hbm, dstbuf)
    _build_slot_map(T, dstbuf, mapb, base_slot, spt, 2 * T, token_only=False)
    bufs = [b0, b1, b2, b3]

    def load(c):
        return pltpu.async_copy(
            ffn_hbm.at[pl.ds(base_slot + c * SROW, SROW)], bufs[c % 4], semg)

    lod = [None] * nr
    sto = [None] * nr
    drained = set()
    for c in range(min(2, nr)):
        lod[c] = load(c)
    for c in range(nr):
        k = c + 2
        if k < nr:
            if k >= 4:
                sto[k - 4].wait()
                drained.add(k - 4)
            lod[k] = load(k)
        lod[c].wait()
        sto[c] = pltpu.async_copy(bufs[c % 4], g_hbm.at[mapb.at[c]], semo)
    for c in range(nr):
        if c not in drained:
            sto[c].wait()


def _combine_call(ffn_pk, dsts_flat, cap, T):
    HP = ffn_pk.shape[1]
    spt = NE * cap // NW
    mesh = plsc.VectorSubcoreMesh(core_axis_name="c", subcore_axis_name="s",
                                  num_cores=NC, num_subcores=NS)
    return pl.kernel(
        functools.partial(_combine_body, T, HP, cap),
        out_type=jax.ShapeDtypeStruct((2 * T + 8, HP), jnp.int32),
        mesh=mesh,
        compiler_params=pltpu.CompilerParams(needs_layout_passes=False),
        scratch_types=[
            pltpu.VMEM((2 * T,), jnp.int32),
            pltpu.VMEM((spt // SROW, SROW), jnp.int32),
            pltpu.VMEM((SROW, HP), jnp.int32),
            pltpu.VMEM((SROW, HP), jnp.int32),
            pltpu.VMEM((SROW, HP), jnp.int32),
            pltpu.VMEM((SROW, HP), jnp.int32),
            pltpu.SemaphoreType.DMA,
            pltpu.SemaphoreType.DMA,
        ],
    )(ffn_pk, dsts_flat)


def _wsum_body(g0_ref, g1_ref, w0_ref, w1_ref, out_ref):
    w0 = w0_ref[:, 0:1]
    w1 = w1_ref[:, 0:1]
    a0 = jnp.where(w0 > 0, _unpack_bf16(g0_ref[...]) * w0, 0.0)
    a1 = jnp.where(w1 > 0, _unpack_bf16(g1_ref[...]) * w1, 0.0)
    out_ref[...] = a0 + a1


def _wsum_call(gpk, w0x, w1x, T):
    HP = gpk.shape[1]
    H = 2 * HP
    TB = 512
    nb = T // TB
    return pl.pallas_call(
        _wsum_body,
        grid=(nb,),
        in_specs=[
            pl.BlockSpec((TB, HP), lambda t: (t, 0)),
            pl.BlockSpec((TB, HP), lambda t, n=nb: (t + n, 0)),
            pl.BlockSpec((TB, SCL), lambda t: (t, 0)),
            pl.BlockSpec((TB, SCL), lambda t: (t, 0)),
        ],
        out_specs=pl.BlockSpec((TB, H), lambda t: (t, 0)),
        out_shape=jax.ShapeDtypeStruct((T, H), jnp.float32),
    )(gpk, gpk, w0x, w1x)


def kernel(x, router_w, gate_w, gate_b, up_w, up_b, down_w, down_b):
    bsz, seq, H = x.shape
    T = bsz * seq
    cap = max(1, math.ceil(CAP_F * T / NE))
    x_flat = x.reshape(T, H)

    rw_pad = jnp.pad(router_w, ((0, 0), (0, LANES - NE)))
    ltri = jnp.asarray(np.tril(np.ones((128, 128), np.float32)))

    dsts, w0x, w1x, aux, z, xpk = _router_call(x_flat, rw_pad, ltri, cap)
    dsts_flat = dsts.reshape(-1)

    disp_pk = _dispatch_call(xpk, dsts_flat, cap)
    ffn_pk = _ffn_call(disp_pk, gate_w, gate_b, up_w, up_b, down_w, down_b,
                       cap)
    gpk = _combine_call(ffn_pk, dsts_flat, cap, T)
    out = _wsum_call(gpk, w0x, w1x, T)

    return (out.reshape(bsz, seq, H), aux.reshape(()), z.reshape(()))
